# Initial kernel scaffold; baseline (speedup 1.0000x reference)
import jax
import jax.numpy as jnp
from jax.experimental import pallas as pl


def kernel(x_user, x_movie, edge_index_rates, edge_index_rev, edge_label_index, W_l_rates_0, b_l_rates_0, W_r_rates_0, W_l_rev_0, b_l_rev_0, W_r_rev_0, W_l_rates_1, b_l_rates_1, W_r_rates_1, W_l_rev_1, b_l_rev_1, W_r_rev_1):
    raise NotImplementedError("write your pallas kernel here")


# speedup vs baseline: 2.3235x; 2.3235x over previous
import functools

import jax
import jax.numpy as jnp
from jax import lax
from jax.experimental import pallas as pl
from jax.experimental.pallas import tpu as pltpu
from jax.experimental.pallas import tpu_sc as plsc

N_NODES = 10000
D = 128
E = 320000
E_LABEL = 100000
NC = 2
NS = 16
NW = NC * NS
CHUNK = 128

N_CHUNKS = 80
PER_W = N_CHUNKS * CHUNK
E_PAD = PER_W * NW

L_PER_W = ((E_LABEL // NW) + CHUNK - 1) // CHUNK * CHUNK
L_PAD = L_PER_W * NW
L_CHUNKS = L_PER_W // CHUNK

ACC_ROWS = 10240
ZROWS = ACC_ROWS // NS

_MESH = plsc.VectorSubcoreMesh(core_axis_name="c", subcore_axis_name="s")


def _segsum_body(x_hbm, src_hbm, dst_hbm, zacc_hbm,
                 acc_hbm, idx_s, idx_d, rows, acc_sh):
    cid = lax.axis_index("c")
    sid = lax.axis_index("s")
    wid = sid * NC + cid

    z0 = sid * ZROWS
    pltpu.sync_copy(zacc_hbm, acc_sh.at[pl.ds(z0, ZROWS)])

    r0 = wid * N_CHUNKS
    pltpu.sync_copy(src_hbm.at[pl.ds(r0, N_CHUNKS)], idx_s)
    pltpu.sync_copy(dst_hbm.at[pl.ds(r0, N_CHUNKS)], idx_d)
    plsc.subcore_barrier()

    @pl.loop(0, N_CHUNKS)
    def _(k):
        pltpu.sync_copy(x_hbm.at[idx_s.at[k]], rows)
        pltpu.sync_copy(rows, acc_sh.at[idx_d.at[k]], add=True)

    plsc.subcore_barrier()
    pltpu.sync_copy(acc_sh.at[pl.ds(z0, ZROWS)],
                    acc_hbm.at[cid, pl.ds(z0, ZROWS)])


_SEGSUM = pl.kernel(
    _segsum_body,
    out_type=jax.ShapeDtypeStruct((NC, ACC_ROWS, D), jnp.float32),
    mesh=_MESH,
    scratch_types=[
        pltpu.VMEM((N_CHUNKS, CHUNK), jnp.int32),
        pltpu.VMEM((N_CHUNKS, CHUNK), jnp.int32),
        pltpu.VMEM((CHUNK, D), jnp.float32),
        pltpu.VMEM_SHARED((ACC_ROWS, D), jnp.float32),
    ])


def _count_body(dst_hbm, zcnt_hbm, ones_hbm,
                cnt_hbm, idx_d, ones_v, cnt_sh):
    cid = lax.axis_index("c")
    sid = lax.axis_index("s")
    wid = sid * NC + cid

    z0 = sid * ZROWS
    pltpu.sync_copy(zcnt_hbm, cnt_sh.at[pl.ds(z0, ZROWS)])
    pltpu.sync_copy(ones_hbm, ones_v)
    r0 = wid * N_CHUNKS
    pltpu.sync_copy(dst_hbm.at[pl.ds(r0, N_CHUNKS)], idx_d)
    plsc.subcore_barrier()

    @pl.loop(0, N_CHUNKS)
    def _(k):
        pltpu.sync_copy(ones_v, cnt_sh.at[idx_d.at[k]], add=True)

    plsc.subcore_barrier()
    pltpu.sync_copy(cnt_sh.at[pl.ds(z0, ZROWS)],
                    cnt_hbm.at[cid, pl.ds(z0, ZROWS)])


_COUNT = pl.kernel(
    _count_body,
    out_type=jax.ShapeDtypeStruct((NC, ACC_ROWS, D), jnp.float32),
    mesh=_MESH,
    scratch_types=[
        pltpu.VMEM((N_CHUNKS, CHUNK), jnp.int32),
        pltpu.VMEM((CHUNK, D), jnp.float32),
        pltpu.VMEM_SHARED((ACC_ROWS, D), jnp.float32),
    ])


def _gather_labels(xu, xm, lu, lm):
    out_type = (jax.ShapeDtypeStruct((L_PAD, D), jnp.float32),
                jax.ShapeDtypeStruct((L_PAD, D), jnp.float32))
    scratch = [
        pltpu.VMEM((L_PER_W,), jnp.int32),
        pltpu.VMEM((L_PER_W,), jnp.int32),
        pltpu.VMEM((CHUNK, D), jnp.float32),
        pltpu.VMEM((CHUNK, D), jnp.float32),
    ]

    def body(xu_hbm, xm_hbm, lu_hbm, lm_hbm, fu_hbm, fm_hbm,
             idx_u, idx_m, rows_u, rows_m):
        cid = lax.axis_index("c")
        sid = lax.axis_index("s")
        wid = sid * NC + cid
        base = wid * L_PER_W
        pltpu.sync_copy(lu_hbm.at[pl.ds(base, L_PER_W)], idx_u)
        pltpu.sync_copy(lm_hbm.at[pl.ds(base, L_PER_W)], idx_m)

        @pl.loop(0, L_CHUNKS)
        def _(k):
            off = base + k * CHUNK
            pltpu.sync_copy(xu_hbm.at[idx_u.at[pl.ds(k * CHUNK, CHUNK)]], rows_u)
            pltpu.sync_copy(xm_hbm.at[idx_m.at[pl.ds(k * CHUNK, CHUNK)]], rows_m)
            pltpu.sync_copy(rows_u, fu_hbm.at[pl.ds(off, CHUNK)])
            pltpu.sync_copy(rows_m, fm_hbm.at[pl.ds(off, CHUNK)])

    f = pl.kernel(body, out_type=out_type, mesh=_MESH, scratch_types=scratch)
    return f(xu, xm, lu, lm)


_BLK = 400


def _combine_body(acc_ref, cnt_ref, x_ref, wl_ref, b_ref, wr_ref, o_ref):
    a = acc_ref[0] + acc_ref[1]
    c = cnt_ref[0][:, :1] + cnt_ref[1][:, :1]
    mean = a * (1.0 / jnp.maximum(c, 1.0))
    o = (jnp.dot(mean, wl_ref[...], preferred_element_type=jnp.float32)
         + jnp.dot(x_ref[...], wr_ref[...], preferred_element_type=jnp.float32)
         + b_ref[...])
    o_ref[...] = jnp.where(o >= 0.0, o, 0.01 * o)


def _combine(acc, cnt, x_dst, w_l, b_l, w_r):
    return pl.pallas_call(
        _combine_body,
        out_shape=jax.ShapeDtypeStruct((N_NODES, D), jnp.float32),
        grid=(N_NODES // _BLK,),
        in_specs=[
            pl.BlockSpec((NC, _BLK, D), lambda i: (0, i, 0)),
            pl.BlockSpec((NC, _BLK, D), lambda i: (0, i, 0)),
            pl.BlockSpec((_BLK, D), lambda i: (i, 0)),
            pl.BlockSpec((D, D), lambda i: (0, 0)),
            pl.BlockSpec((1, D), lambda i: (0, 0)),
            pl.BlockSpec((D, D), lambda i: (0, 0)),
        ],
        out_specs=pl.BlockSpec((_BLK, D), lambda i: (i, 0)),
    )(acc, cnt, x_dst, w_l, b_l.reshape(1, D), w_r)


_DBLK = 1024


def _dot_body(u_ref, m_ref, o_ref):
    o_ref[...] = jnp.sum(u_ref[...] * m_ref[...], axis=1)


def _dot(fu, fm):
    return pl.pallas_call(
        _dot_body,
        out_shape=jax.ShapeDtypeStruct((L_PAD,), jnp.float32),
        grid=(L_PAD // _DBLK,),
        in_specs=[pl.BlockSpec((_DBLK, D), lambda i: (i, 0)),
                  pl.BlockSpec((_DBLK, D), lambda i: (i, 0))],
        out_specs=pl.BlockSpec((_DBLK,), lambda i: (i,)),
    )(fu, fm)


def _pad_edges(ei):
    src = ei[0].astype(jnp.int32)
    dst = ei[1].astype(jnp.int32)
    pad = E_PAD - E
    src = jnp.concatenate([src, jnp.zeros((pad,), jnp.int32)])
    dst = jnp.concatenate([dst, jnp.full((pad,), N_NODES, jnp.int32)])
    return src.reshape(-1, CHUNK), dst.reshape(-1, CHUNK)


def kernel(x_user, x_movie, edge_index_rates, edge_index_rev, edge_label_index,
           W_l_rates_0, b_l_rates_0, W_r_rates_0, W_l_rev_0, b_l_rev_0, W_r_rev_0,
           W_l_rates_1, b_l_rates_1, W_r_rates_1, W_l_rev_1, b_l_rev_1, W_r_rev_1):
    src_r, dst_r = _pad_edges(edge_index_rates)
    src_v, dst_v = _pad_edges(edge_index_rev)
    lpad = L_PAD - E_LABEL
    lu = jnp.concatenate([edge_label_index[0].astype(jnp.int32),
                          jnp.zeros((lpad,), jnp.int32)])
    lm = jnp.concatenate([edge_label_index[1].astype(jnp.int32),
                          jnp.zeros((lpad,), jnp.int32)])

    zacc = jnp.zeros((ZROWS, D), jnp.float32)
    ones = jnp.ones((CHUNK, D), jnp.float32)

    cnt_m = _COUNT(dst_r, zacc, ones)
    cnt_u = _COUNT(dst_v, zacc, ones)

    acc_m = _SEGSUM(x_user, src_r, dst_r, zacc)
    acc_u = _SEGSUM(x_movie, src_v, dst_v, zacc)
    xm1 = _combine(acc_m, cnt_m, x_movie, W_l_rates_0, b_l_rates_0, W_r_rates_0)
    xu1 = _combine(acc_u, cnt_u, x_user, W_l_rev_0, b_l_rev_0, W_r_rev_0)

    acc_m1 = _SEGSUM(xu1, src_r, dst_r, zacc)
    acc_u1 = _SEGSUM(xm1, src_v, dst_v, zacc)
    xm2 = _combine(acc_m1, cnt_m, xm1, W_l_rates_1, b_l_rates_1, W_r_rates_1)
    xu2 = _combine(acc_u1, cnt_u, xu1, W_l_rev_1, b_l_rev_1, W_r_rev_1)

    fu, fm = _gather_labels(xu2, xm2, lu, lm)
    return _dot(fu, fm)[:E_LABEL]


# speedup vs baseline: 2.5574x; 1.1006x over previous
import functools

import jax
import jax.numpy as jnp
from jax import lax
from jax.experimental import pallas as pl
from jax.experimental.pallas import tpu as pltpu
from jax.experimental.pallas import tpu_sc as plsc

N_NODES = 10000
D = 128
E = 320000
E_LABEL = 100000
NC = 2
NS = 16
NW = NC * NS
CHUNK = 128

N_CHUNKS = 80
HALF_CH = N_CHUNKS // 2
PER_W = N_CHUNKS * CHUNK
E_PAD = PER_W * NW

L_CHUNKS = 26
L_PER_W = L_CHUNKS * CHUNK
L_PAD = L_PER_W * NW

ACC_ROWS = 10240
ZROWS = ACC_ROWS // NS

_MESH = plsc.VectorSubcoreMesh(core_axis_name="c", subcore_axis_name="s")


def _segsum_body(x_hbm, src_hbm, dst_hbm, zacc_hbm,
                 acc_hbm, idx_s, idx_d, rows0, rows1, sem0, sem1, acc_sh):
    cid = lax.axis_index("c")
    sid = lax.axis_index("s")
    wid = sid * NC + cid

    z0 = sid * ZROWS
    pltpu.sync_copy(zacc_hbm, acc_sh.at[pl.ds(z0, ZROWS)])

    r0 = wid * N_CHUNKS
    plsc.subcore_barrier()

    rows = (rows0, rows1)
    sems = (sem0, sem1)

    @pl.loop(0, 2)
    def _(h):
        pltpu.sync_copy(src_hbm.at[pl.ds(r0 + h * HALF_CH, HALF_CH)], idx_s)
        pltpu.sync_copy(dst_hbm.at[pl.ds(r0 + h * HALF_CH, HALF_CH)], idx_d)
        pltpu.async_copy(x_hbm.at[idx_s.at[0]], rows0, sem0)
        pltpu.async_copy(x_hbm.at[idx_s.at[1]], rows1, sem1)

        @pl.loop(0, HALF_CH, step=2)
        def _(k):
            for b in range(2):
                pltpu.make_async_copy(x_hbm.at[idx_s.at[0]], rows[b], sems[b]).wait()
                pltpu.sync_copy(rows[b], acc_sh.at[idx_d.at[k + b]], add=True)

                @pl.when(k + b + 2 < HALF_CH)
                def _():
                    pltpu.async_copy(x_hbm.at[idx_s.at[k + b + 2]], rows[b], sems[b])

    plsc.subcore_barrier()
    pltpu.sync_copy(acc_sh.at[pl.ds(z0, ZROWS)],
                    acc_hbm.at[cid, pl.ds(z0, ZROWS)])


_SEGSUM = pl.kernel(
    _segsum_body,
    out_type=jax.ShapeDtypeStruct((NC, ACC_ROWS, D), jnp.float32),
    mesh=_MESH,
    scratch_types=[
        pltpu.VMEM((HALF_CH, CHUNK), jnp.int32),
        pltpu.VMEM((HALF_CH, CHUNK), jnp.int32),
        pltpu.VMEM((CHUNK, D), jnp.float32),
        pltpu.VMEM((CHUNK, D), jnp.float32),
        pltpu.SemaphoreType.DMA,
        pltpu.SemaphoreType.DMA,
        pltpu.VMEM_SHARED((ACC_ROWS, D), jnp.float32),
    ])


CW = 128


def _count_body(dst_hbm, zcnt_hbm, ones_hbm,
                cnt_hbm, idx_d, ones_v, cnt_sh):
    cid = lax.axis_index("c")
    sid = lax.axis_index("s")
    wid = sid * NC + cid

    z0 = sid * ZROWS
    pltpu.sync_copy(zcnt_hbm, cnt_sh.at[pl.ds(z0, ZROWS)])
    pltpu.sync_copy(ones_hbm, ones_v)
    r0 = wid * N_CHUNKS
    pltpu.sync_copy(dst_hbm.at[pl.ds(r0, N_CHUNKS)], idx_d)
    plsc.subcore_barrier()

    @pl.loop(0, N_CHUNKS)
    def _(k):
        pltpu.sync_copy(ones_v, cnt_sh.at[idx_d.at[k]], add=True)

    plsc.subcore_barrier()
    pltpu.sync_copy(cnt_sh.at[pl.ds(z0, ZROWS)],
                    cnt_hbm.at[cid, pl.ds(z0, ZROWS)])


_COUNT = pl.kernel(
    _count_body,
    out_type=jax.ShapeDtypeStruct((NC, ACC_ROWS, CW), jnp.float32),
    mesh=_MESH,
    scratch_types=[
        pltpu.VMEM((N_CHUNKS, CHUNK), jnp.int32),
        pltpu.VMEM((CHUNK, CW), jnp.float32),
        pltpu.VMEM_SHARED((ACC_ROWS, CW), jnp.float32),
    ])


def _gather_labels(xu, xm, lu, lm):
    out_type = (jax.ShapeDtypeStruct((L_PAD, D), jnp.float32),
                jax.ShapeDtypeStruct((L_PAD, D), jnp.float32))
    scratch = [
        pltpu.VMEM((L_PER_W,), jnp.int32),
        pltpu.VMEM((L_PER_W,), jnp.int32),
        pltpu.VMEM((CHUNK, D), jnp.float32),
        pltpu.VMEM((CHUNK, D), jnp.float32),
        pltpu.VMEM((CHUNK, D), jnp.float32),
        pltpu.VMEM((CHUNK, D), jnp.float32),
        pltpu.SemaphoreType.DMA,
        pltpu.SemaphoreType.DMA,
        pltpu.SemaphoreType.DMA,
        pltpu.SemaphoreType.DMA,
    ]

    def body(xu_hbm, xm_hbm, lu_hbm, lm_hbm, fu_hbm, fm_hbm,
             idx_u, idx_m, ru0, ru1, rm0, rm1, su0, su1, sm0, sm1):
        cid = lax.axis_index("c")
        sid = lax.axis_index("s")
        wid = sid * NC + cid
        base = wid * L_PER_W
        pltpu.sync_copy(lu_hbm.at[pl.ds(base, L_PER_W)], idx_u)
        pltpu.sync_copy(lm_hbm.at[pl.ds(base, L_PER_W)], idx_m)

        ru = (ru0, ru1)
        rm = (rm0, rm1)
        su = (su0, su1)
        sm = (sm0, sm1)
        for b in range(2):
            pltpu.async_copy(xu_hbm.at[idx_u.at[pl.ds(b * CHUNK, CHUNK)]], ru[b], su[b])
            pltpu.async_copy(xm_hbm.at[idx_m.at[pl.ds(b * CHUNK, CHUNK)]], rm[b], sm[b])

        @pl.loop(0, L_CHUNKS, step=2)
        def _(k):
            for b in range(2):
                off = base + (k + b) * CHUNK
                pltpu.make_async_copy(xu_hbm.at[idx_u.at[pl.ds(0, CHUNK)]], ru[b], su[b]).wait()
                pltpu.make_async_copy(xm_hbm.at[idx_m.at[pl.ds(0, CHUNK)]], rm[b], sm[b]).wait()
                pltpu.sync_copy(ru[b], fu_hbm.at[pl.ds(off, CHUNK)])
                pltpu.sync_copy(rm[b], fm_hbm.at[pl.ds(off, CHUNK)])

                @pl.when(k + b + 2 < L_CHUNKS)
                def _():
                    nxt = (k + b + 2) * CHUNK
                    pltpu.async_copy(xu_hbm.at[idx_u.at[pl.ds(nxt, CHUNK)]], ru[b], su[b])
                    pltpu.async_copy(xm_hbm.at[idx_m.at[pl.ds(nxt, CHUNK)]], rm[b], sm[b])

    f = pl.kernel(body, out_type=out_type, mesh=_MESH, scratch_types=scratch)
    return f(xu, xm, lu, lm)


_BLK = 400


def _combine_body(acc_ref, cnt_ref, x_ref, wl_ref, b_ref, wr_ref, o_ref):
    a = acc_ref[0] + acc_ref[1]
    c = cnt_ref[0][:, :1] + cnt_ref[1][:, :1]
    mean = a * (1.0 / jnp.maximum(c, 1.0))
    o = (jnp.dot(mean, wl_ref[...], preferred_element_type=jnp.float32)
         + jnp.dot(x_ref[...], wr_ref[...], preferred_element_type=jnp.float32)
         + b_ref[...])
    o_ref[...] = jnp.where(o >= 0.0, o, 0.01 * o)


def _combine(acc, cnt, x_dst, w_l, b_l, w_r):
    return pl.pallas_call(
        _combine_body,
        out_shape=jax.ShapeDtypeStruct((N_NODES, D), jnp.float32),
        grid=(N_NODES // _BLK,),
        in_specs=[
            pl.BlockSpec((NC, _BLK, D), lambda i: (0, i, 0)),
            pl.BlockSpec((NC, _BLK, CW), lambda i: (0, i, 0)),
            pl.BlockSpec((_BLK, D), lambda i: (i, 0)),
            pl.BlockSpec((D, D), lambda i: (0, 0)),
            pl.BlockSpec((1, D), lambda i: (0, 0)),
            pl.BlockSpec((D, D), lambda i: (0, 0)),
        ],
        out_specs=pl.BlockSpec((_BLK, D), lambda i: (i, 0)),
    )(acc, cnt, x_dst, w_l, b_l.reshape(1, D), w_r)


_DBLK = 1024


def _dot_body(u_ref, m_ref, o_ref):
    o_ref[...] = jnp.sum(u_ref[...] * m_ref[...], axis=1)


def _dot(fu, fm):
    return pl.pallas_call(
        _dot_body,
        out_shape=jax.ShapeDtypeStruct((L_PAD,), jnp.float32),
        grid=(L_PAD // _DBLK,),
        in_specs=[pl.BlockSpec((_DBLK, D), lambda i: (i, 0)),
                  pl.BlockSpec((_DBLK, D), lambda i: (i, 0))],
        out_specs=pl.BlockSpec((_DBLK,), lambda i: (i,)),
    )(fu, fm)


def _pad_edges(ei):
    src = ei[0].astype(jnp.int32)
    dst = ei[1].astype(jnp.int32)
    pad = E_PAD - E
    src = jnp.concatenate([src, jnp.zeros((pad,), jnp.int32)])
    dst = jnp.concatenate([dst, jnp.full((pad,), N_NODES, jnp.int32)])
    return src.reshape(-1, CHUNK), dst.reshape(-1, CHUNK)


def kernel(x_user, x_movie, edge_index_rates, edge_index_rev, edge_label_index,
           W_l_rates_0, b_l_rates_0, W_r_rates_0, W_l_rev_0, b_l_rev_0, W_r_rev_0,
           W_l_rates_1, b_l_rates_1, W_r_rates_1, W_l_rev_1, b_l_rev_1, W_r_rev_1):
    src_r, dst_r = _pad_edges(edge_index_rates)
    src_v, dst_v = _pad_edges(edge_index_rev)
    lpad = L_PAD - E_LABEL
    lu = jnp.concatenate([edge_label_index[0].astype(jnp.int32),
                          jnp.zeros((lpad,), jnp.int32)])
    lm = jnp.concatenate([edge_label_index[1].astype(jnp.int32),
                          jnp.zeros((lpad,), jnp.int32)])

    zacc = jnp.zeros((ZROWS, D), jnp.float32)
    zcnt = jnp.zeros((ZROWS, CW), jnp.float32)
    ones = jnp.ones((CHUNK, CW), jnp.float32)

    cnt_m = _COUNT(dst_r, zcnt, ones)
    cnt_u = _COUNT(dst_v, zcnt, ones)

    acc_m = _SEGSUM(x_user, src_r, dst_r, zacc)
    acc_u = _SEGSUM(x_movie, src_v, dst_v, zacc)
    xm1 = _combine(acc_m, cnt_m, x_movie, W_l_rates_0, b_l_rates_0, W_r_rates_0)
    xu1 = _combine(acc_u, cnt_u, x_user, W_l_rev_0, b_l_rev_0, W_r_rev_0)

    acc_m1 = _SEGSUM(xu1, src_r, dst_r, zacc)
    acc_u1 = _SEGSUM(xm1, src_v, dst_v, zacc)
    xm2 = _combine(acc_m1, cnt_m, xm1, W_l_rates_1, b_l_rates_1, W_r_rates_1)
    xu2 = _combine(acc_u1, cnt_u, xu1, W_l_rev_1, b_l_rev_1, W_r_rev_1)

    fu, fm = _gather_labels(xu2, xm2, lu, lm)
    return _dot(fu, fm)[:E_LABEL]


# speedup vs baseline: 3.4336x; 1.3426x over previous
import functools

import jax
import jax.numpy as jnp
from jax import lax
from jax.experimental import pallas as pl
from jax.experimental.pallas import tpu as pltpu
from jax.experimental.pallas import tpu_sc as plsc

N_NODES = 10000
D = 128
E = 320000
E_LABEL = 100000
NC = 2
NS = 16
NW = NC * NS
CHUNK = 128

TOT_CH = 2560
E_PAD = TOT_CH * CHUNK
CH_PER_W = TOT_CH // NS
QCH = 40
N_STAGES = CH_PER_W // QCH

L_CHUNKS = 26
L_PER_W = L_CHUNKS * CHUNK
L_PAD = L_PER_W * NW

ACC_ROWS = 10240
ZROWS = ACC_ROWS // NS

_MESH = plsc.VectorSubcoreMesh(core_axis_name="c", subcore_axis_name="s")


def _segsum_body(xa_hbm, xb_hbm, srcs_hbm, dsts_hbm, zacc_hbm,
                 acc_hbm, idx_s, idx_d, rows0, rows1, sem0, sem1, acc_sh):
    cid = lax.axis_index("c")
    sid = lax.axis_index("s")

    z0 = sid * ZROWS
    pltpu.sync_copy(zacc_hbm, acc_sh.at[pl.ds(z0, ZROWS)])

    r0 = sid * CH_PER_W
    plsc.subcore_barrier()

    rows = (rows0, rows1)
    sems = (sem0, sem1)

    def run(x_hbm):
        @pl.loop(0, N_STAGES)
        def _(h):
            pltpu.sync_copy(srcs_hbm.at[cid, pl.ds(r0 + h * QCH, QCH)], idx_s)
            pltpu.sync_copy(dsts_hbm.at[cid, pl.ds(r0 + h * QCH, QCH)], idx_d)
            pltpu.async_copy(x_hbm.at[idx_s.at[0]], rows0, sem0)
            pltpu.async_copy(x_hbm.at[idx_s.at[1]], rows1, sem1)

            @pl.loop(0, QCH, step=2)
            def _(k):
                for b in range(2):
                    pltpu.make_async_copy(x_hbm.at[idx_s.at[0]], rows[b], sems[b]).wait()
                    pltpu.sync_copy(rows[b], acc_sh.at[idx_d.at[k + b]], add=True)

                    @pl.when(k + b + 2 < QCH)
                    def _():
                        pltpu.async_copy(x_hbm.at[idx_s.at[k + b + 2]], rows[b], sems[b])

    @pl.when(cid == 0)
    def _():
        run(xa_hbm)

    @pl.when(cid == 1)
    def _():
        run(xb_hbm)

    plsc.subcore_barrier()
    pltpu.sync_copy(acc_sh.at[pl.ds(z0, ZROWS)],
                    acc_hbm.at[cid, pl.ds(z0, ZROWS)])


_SEGSUM = pl.kernel(
    _segsum_body,
    out_type=jax.ShapeDtypeStruct((NC, ACC_ROWS, D), jnp.float32),
    mesh=_MESH,
    scratch_types=[
        pltpu.VMEM((QCH, CHUNK), jnp.int32),
        pltpu.VMEM((QCH, CHUNK), jnp.int32),
        pltpu.VMEM((CHUNK, D), jnp.float32),
        pltpu.VMEM((CHUNK, D), jnp.float32),
        pltpu.SemaphoreType.DMA,
        pltpu.SemaphoreType.DMA,
        pltpu.VMEM_SHARED((ACC_ROWS, D), jnp.float32),
    ])


CW = 128


def _count_body(dsts_hbm, zcnt_hbm, ones_hbm,
                cnt_hbm, idx_d, ones_v, cnt_sh):
    cid = lax.axis_index("c")
    sid = lax.axis_index("s")

    z0 = sid * ZROWS
    pltpu.sync_copy(zcnt_hbm, cnt_sh.at[pl.ds(z0, ZROWS)])
    pltpu.sync_copy(ones_hbm, ones_v)
    r0 = sid * CH_PER_W
    pltpu.sync_copy(dsts_hbm.at[cid, pl.ds(r0, CH_PER_W)], idx_d)
    plsc.subcore_barrier()

    @pl.loop(0, CH_PER_W)
    def _(k):
        pltpu.sync_copy(ones_v, cnt_sh.at[idx_d.at[k]], add=True)

    plsc.subcore_barrier()
    pltpu.sync_copy(cnt_sh.at[pl.ds(z0, ZROWS)],
                    cnt_hbm.at[cid, pl.ds(z0, ZROWS)])


_COUNT = pl.kernel(
    _count_body,
    out_type=jax.ShapeDtypeStruct((NC, ACC_ROWS, CW), jnp.float32),
    mesh=_MESH,
    scratch_types=[
        pltpu.VMEM((CH_PER_W, CHUNK), jnp.int32),
        pltpu.VMEM((CHUNK, CW), jnp.float32),
        pltpu.VMEM_SHARED((ACC_ROWS, CW), jnp.float32),
    ])


def _gather_labels(xu, xm, lu, lm):
    out_type = (jax.ShapeDtypeStruct((L_PAD, D), jnp.float32),
                jax.ShapeDtypeStruct((L_PAD, D), jnp.float32))
    scratch = [
        pltpu.VMEM((L_PER_W,), jnp.int32),
        pltpu.VMEM((L_PER_W,), jnp.int32),
        pltpu.VMEM((CHUNK, D), jnp.float32),
        pltpu.VMEM((CHUNK, D), jnp.float32),
        pltpu.VMEM((CHUNK, D), jnp.float32),
        pltpu.VMEM((CHUNK, D), jnp.float32),
        pltpu.SemaphoreType.DMA,
        pltpu.SemaphoreType.DMA,
        pltpu.SemaphoreType.DMA,
        pltpu.SemaphoreType.DMA,
    ]

    def body(xu_hbm, xm_hbm, lu_hbm, lm_hbm, fu_hbm, fm_hbm,
             idx_u, idx_m, ru0, ru1, rm0, rm1, su0, su1, sm0, sm1):
        cid = lax.axis_index("c")
        sid = lax.axis_index("s")
        wid = sid * NC + cid
        base = wid * L_PER_W
        pltpu.sync_copy(lu_hbm.at[pl.ds(base, L_PER_W)], idx_u)
        pltpu.sync_copy(lm_hbm.at[pl.ds(base, L_PER_W)], idx_m)

        ru = (ru0, ru1)
        rm = (rm0, rm1)
        su = (su0, su1)
        sm = (sm0, sm1)
        for b in range(2):
            pltpu.async_copy(xu_hbm.at[idx_u.at[pl.ds(b * CHUNK, CHUNK)]], ru[b], su[b])
            pltpu.async_copy(xm_hbm.at[idx_m.at[pl.ds(b * CHUNK, CHUNK)]], rm[b], sm[b])

        @pl.loop(0, L_CHUNKS, step=2)
        def _(k):
            for b in range(2):
                off = base + (k + b) * CHUNK
                pltpu.make_async_copy(xu_hbm.at[idx_u.at[pl.ds(0, CHUNK)]], ru[b], su[b]).wait()
                pltpu.make_async_copy(xm_hbm.at[idx_m.at[pl.ds(0, CHUNK)]], rm[b], sm[b]).wait()
                pltpu.sync_copy(ru[b], fu_hbm.at[pl.ds(off, CHUNK)])
                pltpu.sync_copy(rm[b], fm_hbm.at[pl.ds(off, CHUNK)])

                @pl.when(k + b + 2 < L_CHUNKS)
                def _():
                    nxt = (k + b + 2) * CHUNK
                    pltpu.async_copy(xu_hbm.at[idx_u.at[pl.ds(nxt, CHUNK)]], ru[b], su[b])
                    pltpu.async_copy(xm_hbm.at[idx_m.at[pl.ds(nxt, CHUNK)]], rm[b], sm[b])

    f = pl.kernel(body, out_type=out_type, mesh=_MESH, scratch_types=scratch)
    return f(xu, xm, lu, lm)


_BLK = 400


def _combine_body(acc_ref, cnt_ref, x_ref, wl_ref, b_ref, wr_ref, o_ref):
    a = acc_ref[...]
    c = cnt_ref[...][:, :1]
    mean = a * (1.0 / jnp.maximum(c, 1.0))
    o = (jnp.dot(mean, wl_ref[...], preferred_element_type=jnp.float32)
         + jnp.dot(x_ref[...], wr_ref[...], preferred_element_type=jnp.float32)
         + b_ref[...])
    o_ref[...] = jnp.where(o >= 0.0, o, 0.01 * o)


def _combine(acc, cnt, x_dst, w_l, b_l, w_r):
    return pl.pallas_call(
        _combine_body,
        out_shape=jax.ShapeDtypeStruct((N_NODES, D), jnp.float32),
        grid=(N_NODES // _BLK,),
        in_specs=[
            pl.BlockSpec((_BLK, D), lambda i: (i, 0)),
            pl.BlockSpec((_BLK, CW), lambda i: (i, 0)),
            pl.BlockSpec((_BLK, D), lambda i: (i, 0)),
            pl.BlockSpec((D, D), lambda i: (0, 0)),
            pl.BlockSpec((1, D), lambda i: (0, 0)),
            pl.BlockSpec((D, D), lambda i: (0, 0)),
        ],
        out_specs=pl.BlockSpec((_BLK, D), lambda i: (i, 0)),
    )(acc, cnt, x_dst, w_l, b_l.reshape(1, D), w_r)


_DBLK = 1024


def _dot_body(u_ref, m_ref, o_ref):
    o_ref[...] = jnp.sum(u_ref[...] * m_ref[...], axis=1)


def _dot(fu, fm):
    return pl.pallas_call(
        _dot_body,
        out_shape=jax.ShapeDtypeStruct((L_PAD,), jnp.float32),
        grid=(L_PAD // _DBLK,),
        in_specs=[pl.BlockSpec((_DBLK, D), lambda i: (i, 0)),
                  pl.BlockSpec((_DBLK, D), lambda i: (i, 0))],
        out_specs=pl.BlockSpec((_DBLK,), lambda i: (i,)),
    )(fu, fm)


def _pad_edges(ei):
    src = ei[0].astype(jnp.int32)
    dst = ei[1].astype(jnp.int32)
    pad = E_PAD - E
    src = jnp.concatenate([src, jnp.zeros((pad,), jnp.int32)])
    dst = jnp.concatenate([dst, jnp.full((pad,), N_NODES, jnp.int32)])
    return src.reshape(-1, CHUNK), dst.reshape(-1, CHUNK)


def kernel(x_user, x_movie, edge_index_rates, edge_index_rev, edge_label_index,
           W_l_rates_0, b_l_rates_0, W_r_rates_0, W_l_rev_0, b_l_rev_0, W_r_rev_0,
           W_l_rates_1, b_l_rates_1, W_r_rates_1, W_l_rev_1, b_l_rev_1, W_r_rev_1):
    src_r, dst_r = _pad_edges(edge_index_rates)
    src_v, dst_v = _pad_edges(edge_index_rev)
    srcs = jnp.stack([src_r, src_v])
    dsts = jnp.stack([dst_r, dst_v])
    lpad = L_PAD - E_LABEL
    lu = jnp.concatenate([edge_label_index[0].astype(jnp.int32),
                          jnp.zeros((lpad,), jnp.int32)])
    lm = jnp.concatenate([edge_label_index[1].astype(jnp.int32),
                          jnp.zeros((lpad,), jnp.int32)])

    zacc = jnp.zeros((ZROWS, D), jnp.float32)
    zcnt = jnp.zeros((ZROWS, CW), jnp.float32)
    ones = jnp.ones((CHUNK, CW), jnp.float32)

    cnt = _COUNT(dsts, zcnt, ones)

    acc = _SEGSUM(x_user, x_movie, srcs, dsts, zacc)
    xm1 = _combine(acc[0], cnt[0], x_movie, W_l_rates_0, b_l_rates_0, W_r_rates_0)
    xu1 = _combine(acc[1], cnt[1], x_user, W_l_rev_0, b_l_rev_0, W_r_rev_0)

    acc1 = _SEGSUM(xu1, xm1, srcs, dsts, zacc)
    xm2 = _combine(acc1[0], cnt[0], xm1, W_l_rates_1, b_l_rates_1, W_r_rates_1)
    xu2 = _combine(acc1[1], cnt[1], xu1, W_l_rev_1, b_l_rev_1, W_r_rev_1)

    fu, fm = _gather_labels(xu2, xm2, lu, lm)
    return _dot(fu, fm)[:E_LABEL]


# speedup vs baseline: 3.4361x; 1.0007x over previous
import functools

import jax
import jax.numpy as jnp
from jax import lax
from jax.experimental import pallas as pl
from jax.experimental.pallas import tpu as pltpu
from jax.experimental.pallas import tpu_sc as plsc

N_NODES = 10000
D = 128
E = 320000
E_LABEL = 100000
NC = 2
NS = 16
NW = NC * NS
CHUNK = 128

TOT_CH = 2560
E_PAD = TOT_CH * CHUNK
CH_PER_W = TOT_CH // NS
QCH = 40
N_STAGES = CH_PER_W // QCH

L_CHUNKS = 26
L_PER_W = L_CHUNKS * CHUNK
L_PAD = L_PER_W * NW

ACC_ROWS = 10240
ZROWS = ACC_ROWS // NS

_MESH = plsc.VectorSubcoreMesh(core_axis_name="c", subcore_axis_name="s")


def _segsum_body(xa_hbm, xb_hbm, srcs_hbm, dsts_hbm, zacc_hbm,
                 acc_hbm, idx_s, idx_d, rows0, rows1, sem0, sem1, acc_sh):
    cid = lax.axis_index("c")
    sid = lax.axis_index("s")

    z0 = sid * ZROWS
    pltpu.sync_copy(zacc_hbm, rows0)

    @pl.loop(0, ZROWS // CHUNK)
    def _(i):
        pltpu.sync_copy(rows0, acc_sh.at[pl.ds(z0 + i * CHUNK, CHUNK)])

    r0 = sid * CH_PER_W
    plsc.subcore_barrier()

    rows = (rows0, rows1)
    sems = (sem0, sem1)

    def run(x_hbm):
        @pl.loop(0, N_STAGES)
        def _(h):
            pltpu.sync_copy(srcs_hbm.at[cid, pl.ds(r0 + h * QCH, QCH)], idx_s)
            pltpu.sync_copy(dsts_hbm.at[cid, pl.ds(r0 + h * QCH, QCH)], idx_d)
            pltpu.async_copy(x_hbm.at[idx_s.at[0]], rows0, sem0)
            pltpu.async_copy(x_hbm.at[idx_s.at[1]], rows1, sem1)

            @pl.loop(0, QCH, step=2)
            def _(k):
                for b in range(2):
                    pltpu.make_async_copy(x_hbm.at[idx_s.at[0]], rows[b], sems[b]).wait()
                    pltpu.sync_copy(rows[b], acc_sh.at[idx_d.at[k + b]], add=True)

                    @pl.when(k + b + 2 < QCH)
                    def _():
                        pltpu.async_copy(x_hbm.at[idx_s.at[k + b + 2]], rows[b], sems[b])

    @pl.when(cid == 0)
    def _():
        run(xa_hbm)

    @pl.when(cid == 1)
    def _():
        run(xb_hbm)

    plsc.subcore_barrier()
    pltpu.sync_copy(acc_sh.at[pl.ds(z0, ZROWS)],
                    acc_hbm.at[cid, pl.ds(z0, ZROWS)])


_SEGSUM = pl.kernel(
    _segsum_body,
    out_type=jax.ShapeDtypeStruct((NC, ACC_ROWS, D), jnp.float32),
    mesh=_MESH,
    scratch_types=[
        pltpu.VMEM((QCH, CHUNK), jnp.int32),
        pltpu.VMEM((QCH, CHUNK), jnp.int32),
        pltpu.VMEM((CHUNK, D), jnp.float32),
        pltpu.VMEM((CHUNK, D), jnp.float32),
        pltpu.SemaphoreType.DMA,
        pltpu.SemaphoreType.DMA,
        pltpu.VMEM_SHARED((ACC_ROWS, D), jnp.float32),
    ])


CW = 128


def _count_body(dsts_hbm, zcnt_hbm, ones_hbm,
                cnt_hbm, idx_d, ones_v, cnt_sh):
    cid = lax.axis_index("c")
    sid = lax.axis_index("s")

    z0 = sid * ZROWS
    pltpu.sync_copy(zcnt_hbm, ones_v)

    @pl.loop(0, ZROWS // CHUNK)
    def _(i):
        pltpu.sync_copy(ones_v, cnt_sh.at[pl.ds(z0 + i * CHUNK, CHUNK)])

    pltpu.sync_copy(ones_hbm, ones_v)
    r0 = sid * CH_PER_W
    pltpu.sync_copy(dsts_hbm.at[cid, pl.ds(r0, CH_PER_W)], idx_d)
    plsc.subcore_barrier()

    @pl.loop(0, CH_PER_W)
    def _(k):
        pltpu.sync_copy(ones_v, cnt_sh.at[idx_d.at[k]], add=True)

    plsc.subcore_barrier()
    pltpu.sync_copy(cnt_sh.at[pl.ds(z0, ZROWS)],
                    cnt_hbm.at[cid, pl.ds(z0, ZROWS)])


_COUNT = pl.kernel(
    _count_body,
    out_type=jax.ShapeDtypeStruct((NC, ACC_ROWS, CW), jnp.float32),
    mesh=_MESH,
    scratch_types=[
        pltpu.VMEM((CH_PER_W, CHUNK), jnp.int32),
        pltpu.VMEM((CHUNK, CW), jnp.float32),
        pltpu.VMEM_SHARED((ACC_ROWS, CW), jnp.float32),
    ])


def _gather_labels(xu, xm, lu, lm):
    out_type = (jax.ShapeDtypeStruct((L_PAD, D), jnp.float32),
                jax.ShapeDtypeStruct((L_PAD, D), jnp.float32))
    scratch = [
        pltpu.VMEM((L_PER_W,), jnp.int32),
        pltpu.VMEM((L_PER_W,), jnp.int32),
        pltpu.VMEM((CHUNK, D), jnp.float32),
        pltpu.VMEM((CHUNK, D), jnp.float32),
        pltpu.VMEM((CHUNK, D), jnp.float32),
        pltpu.VMEM((CHUNK, D), jnp.float32),
        pltpu.SemaphoreType.DMA,
        pltpu.SemaphoreType.DMA,
        pltpu.SemaphoreType.DMA,
        pltpu.SemaphoreType.DMA,
    ]

    def body(xu_hbm, xm_hbm, lu_hbm, lm_hbm, fu_hbm, fm_hbm,
             idx_u, idx_m, ru0, ru1, rm0, rm1, su0, su1, sm0, sm1):
        cid = lax.axis_index("c")
        sid = lax.axis_index("s")
        wid = sid * NC + cid
        base = wid * L_PER_W
        pltpu.sync_copy(lu_hbm.at[pl.ds(base, L_PER_W)], idx_u)
        pltpu.sync_copy(lm_hbm.at[pl.ds(base, L_PER_W)], idx_m)

        ru = (ru0, ru1)
        rm = (rm0, rm1)
        su = (su0, su1)
        sm = (sm0, sm1)
        for b in range(2):
            pltpu.async_copy(xu_hbm.at[idx_u.at[pl.ds(b * CHUNK, CHUNK)]], ru[b], su[b])
            pltpu.async_copy(xm_hbm.at[idx_m.at[pl.ds(b * CHUNK, CHUNK)]], rm[b], sm[b])

        @pl.loop(0, L_CHUNKS, step=2)
        def _(k):
            for b in range(2):
                off = base + (k + b) * CHUNK
                pltpu.make_async_copy(xu_hbm.at[idx_u.at[pl.ds(0, CHUNK)]], ru[b], su[b]).wait()
                pltpu.make_async_copy(xm_hbm.at[idx_m.at[pl.ds(0, CHUNK)]], rm[b], sm[b]).wait()
                pltpu.sync_copy(ru[b], fu_hbm.at[pl.ds(off, CHUNK)])
                pltpu.sync_copy(rm[b], fm_hbm.at[pl.ds(off, CHUNK)])

                @pl.when(k + b + 2 < L_CHUNKS)
                def _():
                    nxt = (k + b + 2) * CHUNK
                    pltpu.async_copy(xu_hbm.at[idx_u.at[pl.ds(nxt, CHUNK)]], ru[b], su[b])
                    pltpu.async_copy(xm_hbm.at[idx_m.at[pl.ds(nxt, CHUNK)]], rm[b], sm[b])

    f = pl.kernel(body, out_type=out_type, mesh=_MESH, scratch_types=scratch)
    return f(xu, xm, lu, lm)


_BLK = 400


def _combine_body(acc_ref, cnt_ref, x_ref, wl_ref, b_ref, wr_ref, o_ref):
    a = acc_ref[...]
    c = cnt_ref[...][:, :1]
    mean = a * (1.0 / jnp.maximum(c, 1.0))
    o = (jnp.dot(mean, wl_ref[...], preferred_element_type=jnp.float32)
         + jnp.dot(x_ref[...], wr_ref[...], preferred_element_type=jnp.float32)
         + b_ref[...])
    o_ref[...] = jnp.where(o >= 0.0, o, 0.01 * o)


def _combine(acc, cnt, x_dst, w_l, b_l, w_r):
    return pl.pallas_call(
        _combine_body,
        out_shape=jax.ShapeDtypeStruct((N_NODES, D), jnp.float32),
        grid=(N_NODES // _BLK,),
        in_specs=[
            pl.BlockSpec((_BLK, D), lambda i: (i, 0)),
            pl.BlockSpec((_BLK, CW), lambda i: (i, 0)),
            pl.BlockSpec((_BLK, D), lambda i: (i, 0)),
            pl.BlockSpec((D, D), lambda i: (0, 0)),
            pl.BlockSpec((1, D), lambda i: (0, 0)),
            pl.BlockSpec((D, D), lambda i: (0, 0)),
        ],
        out_specs=pl.BlockSpec((_BLK, D), lambda i: (i, 0)),
    )(acc, cnt, x_dst, w_l, b_l.reshape(1, D), w_r)


_DBLK = 1024


def _dot_body(u_ref, m_ref, o_ref):
    u = u_ref[...].astype(jnp.float32)
    m = m_ref[...].astype(jnp.float32)
    o_ref[...] = jnp.sum(u * m, axis=1)


def _dot(fu, fm):
    return pl.pallas_call(
        _dot_body,
        out_shape=jax.ShapeDtypeStruct((L_PAD,), jnp.float32),
        grid=(L_PAD // _DBLK,),
        in_specs=[pl.BlockSpec((_DBLK, D), lambda i: (i, 0)),
                  pl.BlockSpec((_DBLK, D), lambda i: (i, 0))],
        out_specs=pl.BlockSpec((_DBLK,), lambda i: (i,)),
    )(fu, fm)


def _pad_edges(ei):
    src = ei[0].astype(jnp.int32)
    dst = ei[1].astype(jnp.int32)
    pad = E_PAD - E
    src = jnp.concatenate([src, jnp.zeros((pad,), jnp.int32)])
    dst = jnp.concatenate([dst, jnp.full((pad,), N_NODES, jnp.int32)])
    return src.reshape(-1, CHUNK), dst.reshape(-1, CHUNK)


def kernel(x_user, x_movie, edge_index_rates, edge_index_rev, edge_label_index,
           W_l_rates_0, b_l_rates_0, W_r_rates_0, W_l_rev_0, b_l_rev_0, W_r_rev_0,
           W_l_rates_1, b_l_rates_1, W_r_rates_1, W_l_rev_1, b_l_rev_1, W_r_rev_1):
    src_r, dst_r = _pad_edges(edge_index_rates)
    src_v, dst_v = _pad_edges(edge_index_rev)
    srcs = jnp.stack([src_r, src_v])
    dsts = jnp.stack([dst_r, dst_v])
    lpad = L_PAD - E_LABEL
    lu = jnp.concatenate([edge_label_index[0].astype(jnp.int32),
                          jnp.zeros((lpad,), jnp.int32)])
    lm = jnp.concatenate([edge_label_index[1].astype(jnp.int32),
                          jnp.zeros((lpad,), jnp.int32)])

    zacc = jnp.zeros((CHUNK, D), jnp.float32)
    zcnt = jnp.zeros((CHUNK, CW), jnp.float32)
    ones = jnp.ones((CHUNK, CW), jnp.float32)

    cnt = _COUNT(dsts, zcnt, ones)

    acc = _SEGSUM(x_user, x_movie, srcs, dsts, zacc)
    xm1 = _combine(acc[0], cnt[0], x_movie, W_l_rates_0, b_l_rates_0, W_r_rates_0)
    xu1 = _combine(acc[1], cnt[1], x_user, W_l_rev_0, b_l_rev_0, W_r_rev_0)

    acc1 = _SEGSUM(xu1, xm1, srcs, dsts, zacc)
    xm2 = _combine(acc1[0], cnt[0], xm1, W_l_rates_1, b_l_rates_1, W_r_rates_1)
    xu2 = _combine(acc1[1], cnt[1], xu1, W_l_rev_1, b_l_rev_1, W_r_rev_1)

    fu, fm = _gather_labels(xu2, xm2, lu, lm)
    return _dot(fu, fm)[:E_LABEL]
